# baseline (speedup 1.0000x reference)
import functools

import jax
import jax.numpy as jnp
from jax import lax
from jax.experimental import pallas as pl
from jax.experimental.pallas import tpu as pltpu
from jax.experimental.pallas import tpu_sc as plsc

B, S, H, E = 4, 8192, 2048, 64
NC, NS = 2, 16
TCOLS = H // NS
NVT = TCOLS // 16
RSP = 192
S_SC = 3840
RPC = S_SC // NC
NCB = RPC // RSP
NCH = B * NCB
RT = 256
NJ = (S - S_SC) // RT


def _pool_body(hs, out, spmem, slc, accv, hbm_sem, ssem0, ssem1):
    cid = lax.axis_index("c")
    sid = lax.axis_index("s")
    c0 = sid * TCOLS
    ssems = (ssem0, ssem1)

    def hbm_copy(k, slot):
        b = k // NCB
        r0 = cid * RPC + (k % NCB) * RSP
        return pltpu.make_async_copy(
            hs.at[b, pl.ds(r0, RSP), :], spmem.at[slot], hbm_sem)

    def slice_copy(slot):
        return pltpu.make_async_copy(
            spmem.at[slot, :, pl.ds(c0, TCOLS)], slc.at[slot], ssems[slot])

    def compute(k, acc):
        slot = k % 2
        if k % NCB == 0:
            acc = tuple(jnp.zeros((16,), jnp.float32) for _ in range(NVT))

        def row_step(r, a):
            a = tuple(a[i] + slc[slot, 2 * r, pl.ds(i * 16, 16)]
                      for i in range(NVT))
            return tuple(a[i] + slc[slot, 2 * r + 1, pl.ds(i * 16, 16)]
                         for i in range(NVT))

        acc = lax.fori_loop(0, RSP // 2, row_step, acc)
        if k % NCB == NCB - 1:
            b = k // NCB
            for i in range(NVT):
                accv[b, pl.ds(i * 16, 16)] = acc[i]
        return acc

    @pl.when(sid == 0)
    def _():
        hbm_copy(0, 0).start()

    acc = tuple(jnp.zeros((16,), jnp.float32) for _ in range(NVT))
    for k in range(NCH):
        slot = k % 2

        @pl.when(sid == 0)
        def _():
            hbm_copy(k, slot).wait()

        if k >= 1:
            slice_copy((k - 1) % 2).wait()
        plsc.subcore_barrier()
        if k + 1 < NCH:
            @pl.when(sid == 0)
            def _():
                hbm_copy(k + 1, (k + 1) % 2).start()
        slice_copy(slot).start()
        if k >= 1:
            acc = compute(k - 1, acc)
    slice_copy((NCH - 1) % 2).wait()
    compute(NCH - 1, acc)
    pltpu.sync_copy(accv, out.at[cid, :, pl.ds(c0, TCOLS)])


_pool = functools.partial(
    pl.kernel,
    mesh=plsc.VectorSubcoreMesh(core_axis_name="c", subcore_axis_name="s"),
    out_type=jax.ShapeDtypeStruct((NC, B, H), jnp.float32),
    scratch_types=[
        pltpu.VMEM_SHARED((2, RSP, H), jnp.float32),
        pltpu.VMEM((2, RSP, TCOLS), jnp.float32),
        pltpu.VMEM((B, TCOLS), jnp.float32),
        pltpu.SemaphoreType.DMA,
        pltpu.SemaphoreType.DMA,
        pltpu.SemaphoreType.DMA,
    ],
)(_pool_body)


def _tc_pool_body(x_ref, o_ref):
    @pl.when(pl.program_id(0) == 0)
    def _():
        o_ref[...] = jnp.zeros_like(o_ref)

    o_ref[...] += jnp.sum(x_ref[...], axis=1, keepdims=True)


_tc_pool = pl.pallas_call(
    _tc_pool_body,
    grid=(NJ,),
    in_specs=[pl.BlockSpec((B, RT, H), lambda j: (0, S_SC // RT + j, 0))],
    out_specs=pl.BlockSpec((B, 1, H), lambda j: (0, 0, 0)),
    out_shape=jax.ShapeDtypeStruct((B, 1, H), jnp.float32),
)


def _tail_body(ps_sc_ref, ps_tc_ref, w_ref, bias_ref, temp_ref, noise_ref,
               gum_ref, sel_ref, probs_ref):
    psum = ps_tc_ref[:, 0, :]
    for k in range(NC):
        psum = psum + ps_sc_ref[k]
    pooled = psum * (1.0 / S)
    w = w_ref[...]
    logits = lax.dot_general(pooled, w, (((1,), (0,)), ((), ())),
                             preferred_element_type=jnp.float32)
    logits = logits + bias_ref[...] + noise_ref[...]
    t = temp_ref[0, 0]
    x = logits / t
    x = x - jnp.max(x, axis=-1, keepdims=True)
    ex = jnp.exp(x)
    probs = ex / jnp.sum(ex, axis=-1, keepdims=True)
    probs_ref[...] = probs
    y = jnp.log(probs + 1e-20) + gum_ref[...]
    idx = lax.broadcasted_iota(jnp.int32, y.shape, 1)
    m1 = jnp.max(y, axis=-1, keepdims=True)
    i1 = jnp.min(jnp.where(y == m1, idx, E), axis=-1, keepdims=True)
    y2 = jnp.where(idx == i1, -jnp.inf, y)
    m2 = jnp.max(y2, axis=-1, keepdims=True)
    i2 = jnp.min(jnp.where(y2 == m2, idx, E), axis=-1, keepdims=True)
    sel_ref[...] = jnp.concatenate([i1, i2], axis=-1)


_tail = pl.pallas_call(
    _tail_body,
    out_shape=(jax.ShapeDtypeStruct((B, 2), jnp.int32),
               jax.ShapeDtypeStruct((B, E), jnp.float32)),
    in_specs=[
        pl.BlockSpec(memory_space=pltpu.VMEM),
        pl.BlockSpec(memory_space=pltpu.VMEM),
        pl.BlockSpec(memory_space=pltpu.VMEM),
        pl.BlockSpec(memory_space=pltpu.VMEM),
        pl.BlockSpec(memory_space=pltpu.SMEM),
        pl.BlockSpec(memory_space=pltpu.VMEM),
        pl.BlockSpec(memory_space=pltpu.VMEM),
    ],
)


def kernel(hidden_states, W, b, temperature, diversity_bonus):
    noise_key = jax.random.fold_in(jax.random.key(0), 1)
    noise = jax.random.normal(noise_key, (B, E), jnp.float32) * diversity_bonus
    g_key = jax.random.fold_in(jax.random.key(0), 2)
    gumbel = jax.random.gumbel(g_key, (B, E), jnp.float32)
    ps_sc = _pool(hidden_states)
    ps_tc = _tc_pool(hidden_states)
    sel, probs = _tail(ps_sc, ps_tc, W, b.reshape(1, E),
                       temperature.reshape(1, 1), noise, gumbel)
    return sel, probs


# speedup vs baseline: 1.0660x; 1.0660x over previous
import functools

import jax
import jax.numpy as jnp
from jax import lax
from jax.experimental import pallas as pl
from jax.experimental.pallas import tpu as pltpu
from jax.experimental.pallas import tpu_sc as plsc

B, S, H, E = 4, 8192, 2048, 64
NC, NS = 2, 16
NW = NC * NS
SSEG = NW // B
RCH = 16
NPASS = 4
CPP = H // NPASS
NVP = CPP // 16
S_SC = 8192
NR = S_SC // SSEG
NCHUNK = NR // RCH
RT = 256
NJ = (S - S_SC) // RT


def _pool_body(hs, out, buf, accv, sem0, sem1):
    wid = lax.axis_index("s") * NC + lax.axis_index("c")
    b = wid // SSEG
    row0 = (wid % SSEG) * NR

    def copy_in(g, slot, sem):
        return pltpu.make_async_copy(
            hs.at[b, pl.ds(row0 + g * RCH, RCH), :], buf.at[slot], sem)

    zero = jnp.zeros((16,), jnp.float32)
    for i in range(H // 16):
        accv[pl.ds(i * 16, 16)] = zero

    def acc_chunk(slot):
        for h in range(NPASS):
            c0 = h * CPP

            def row_step(r, a):
                return tuple(a[i] + buf[slot, r, pl.ds(c0 + i * 16, 16)]
                             for i in range(NVP))

            a = tuple(accv[pl.ds(c0 + i * 16, 16)] for i in range(NVP))
            a = lax.fori_loop(0, RCH, row_step, a)
            for i in range(NVP):
                accv[pl.ds(c0 + i * 16, 16)] = a[i]

    copy_in(0, 0, sem0).start()

    def chunk_pair(k, carry):
        g = k * 2
        copy_in(g + 1, 1, sem1).start()
        copy_in(g, 0, sem0).wait()
        acc_chunk(0)

        @pl.when(g + 2 < NCHUNK)
        def _():
            copy_in(g + 2, 0, sem0).start()

        copy_in(g + 1, 1, sem1).wait()
        acc_chunk(1)
        return carry

    lax.fori_loop(0, NCHUNK // 2, chunk_pair, 0)
    pltpu.sync_copy(accv, out.at[wid % SSEG, b, :])


_pool = functools.partial(
    pl.kernel,
    mesh=plsc.VectorSubcoreMesh(core_axis_name="c", subcore_axis_name="s"),
    out_type=jax.ShapeDtypeStruct((SSEG, B, H), jnp.float32),
    scratch_types=[
        pltpu.VMEM((2, RCH, H), jnp.float32),
        pltpu.VMEM((H,), jnp.float32),
        pltpu.SemaphoreType.DMA,
        pltpu.SemaphoreType.DMA,
    ],
)(_pool_body)


def _tc_pool_body(x_ref, o_ref):
    @pl.when(pl.program_id(0) == 0)
    def _():
        o_ref[...] = jnp.zeros_like(o_ref)

    o_ref[...] += jnp.sum(x_ref[...], axis=1, keepdims=True)


_tc_pool = pl.pallas_call(
    _tc_pool_body,
    grid=(NJ,),
    in_specs=[pl.BlockSpec((B, RT, H), lambda j: (0, S_SC // RT + j, 0))],
    out_specs=pl.BlockSpec((B, 1, H), lambda j: (0, 0, 0)),
    out_shape=jax.ShapeDtypeStruct((B, 1, H), jnp.float32),
)


def _tail_body(ps_sc_ref, ps_tc_ref, w_ref, bias_ref, temp_ref, noise_ref,
               gum_ref, sel_ref, probs_ref):
    psum = ps_tc_ref[:, 0, :]
    for k in range(SSEG):
        psum = psum + ps_sc_ref[k]
    pooled = psum * (1.0 / S)
    w = w_ref[...]
    logits = lax.dot_general(pooled, w, (((1,), (0,)), ((), ())),
                             preferred_element_type=jnp.float32)
    logits = logits + bias_ref[...] + noise_ref[...]
    t = temp_ref[0, 0]
    x = logits / t
    x = x - jnp.max(x, axis=-1, keepdims=True)
    ex = jnp.exp(x)
    probs = ex / jnp.sum(ex, axis=-1, keepdims=True)
    probs_ref[...] = probs
    y = jnp.log(probs + 1e-20) + gum_ref[...]
    idx = lax.broadcasted_iota(jnp.int32, y.shape, 1)
    m1 = jnp.max(y, axis=-1, keepdims=True)
    i1 = jnp.min(jnp.where(y == m1, idx, E), axis=-1, keepdims=True)
    y2 = jnp.where(idx == i1, -jnp.inf, y)
    m2 = jnp.max(y2, axis=-1, keepdims=True)
    i2 = jnp.min(jnp.where(y2 == m2, idx, E), axis=-1, keepdims=True)
    sel_ref[...] = jnp.concatenate([i1, i2], axis=-1)


_tail = pl.pallas_call(
    _tail_body,
    out_shape=(jax.ShapeDtypeStruct((B, 2), jnp.int32),
               jax.ShapeDtypeStruct((B, E), jnp.float32)),
    in_specs=[
        pl.BlockSpec(memory_space=pltpu.VMEM),
        pl.BlockSpec(memory_space=pltpu.VMEM),
        pl.BlockSpec(memory_space=pltpu.VMEM),
        pl.BlockSpec(memory_space=pltpu.VMEM),
        pl.BlockSpec(memory_space=pltpu.SMEM),
        pl.BlockSpec(memory_space=pltpu.VMEM),
        pl.BlockSpec(memory_space=pltpu.VMEM),
    ],
)


def kernel(hidden_states, W, b, temperature, diversity_bonus):
    noise_key = jax.random.fold_in(jax.random.key(0), 1)
    noise = jax.random.normal(noise_key, (B, E), jnp.float32) * diversity_bonus
    g_key = jax.random.fold_in(jax.random.key(0), 2)
    gumbel = jax.random.gumbel(g_key, (B, E), jnp.float32)
    ps_sc = _pool(hidden_states)
    ps_tc = jnp.zeros((B, 1, H), jnp.float32)
    sel, probs = _tail(ps_sc, ps_tc, W, b.reshape(1, E),
                       temperature.reshape(1, 1), noise, gumbel)
    return sel, probs


# speedup vs baseline: 1.3873x; 1.3014x over previous
import functools

import jax
import jax.numpy as jnp
from jax import lax
from jax.experimental import pallas as pl
from jax.experimental.pallas import tpu as pltpu
from jax.experimental.pallas import tpu_sc as plsc

B, S, H, E = 4, 8192, 2048, 64
NC, NS = 2, 16
NW = NC * NS
SSEG = NW // B
RCH = 24
NPASS = 4
CPP = H // NPASS
NVP = CPP // 16
S_SC = 768
NR = S_SC // SSEG
NCHUNK = NR // RCH
RT = 256
NJ = (S - S_SC) // RT


def _pool_body(hs, out, buf, accv, sem0, sem1):
    wid = lax.axis_index("s") * NC + lax.axis_index("c")
    b = wid // SSEG
    row0 = (wid % SSEG) * NR

    def copy_in(g, slot, sem):
        return pltpu.make_async_copy(
            hs.at[b, pl.ds(row0 + g * RCH, RCH), :], buf.at[slot], sem)

    zero = jnp.zeros((16,), jnp.float32)
    for i in range(H // 16):
        accv[pl.ds(i * 16, 16)] = zero

    def acc_chunk(slot):
        for h in range(NPASS):
            c0 = h * CPP

            def row_step(r, a):
                return tuple(a[i] + buf[slot, r, pl.ds(c0 + i * 16, 16)]
                             for i in range(NVP))

            a = tuple(accv[pl.ds(c0 + i * 16, 16)] for i in range(NVP))
            a = lax.fori_loop(0, RCH, row_step, a)
            for i in range(NVP):
                accv[pl.ds(c0 + i * 16, 16)] = a[i]

    copy_in(0, 0, sem0).start()

    def chunk_pair(k, carry):
        g = k * 2
        copy_in(g + 1, 1, sem1).start()
        copy_in(g, 0, sem0).wait()
        acc_chunk(0)

        @pl.when(g + 2 < NCHUNK)
        def _():
            copy_in(g + 2, 0, sem0).start()

        copy_in(g + 1, 1, sem1).wait()
        acc_chunk(1)
        return carry

    lax.fori_loop(0, NCHUNK // 2, chunk_pair, 0)
    pltpu.sync_copy(accv, out.at[wid % SSEG, b, :])


_pool = functools.partial(
    pl.kernel,
    mesh=plsc.VectorSubcoreMesh(core_axis_name="c", subcore_axis_name="s"),
    out_type=jax.ShapeDtypeStruct((SSEG, B, H), jnp.float32),
    scratch_types=[
        pltpu.VMEM((2, RCH, H), jnp.float32),
        pltpu.VMEM((H,), jnp.float32),
        pltpu.SemaphoreType.DMA,
        pltpu.SemaphoreType.DMA,
    ],
)(_pool_body)


def _tc_pool_body(x_ref, o_ref):
    @pl.when(pl.program_id(0) == 0)
    def _():
        o_ref[...] = jnp.zeros_like(o_ref)

    o_ref[...] += jnp.sum(x_ref[...], axis=1, keepdims=True)


_tc_pool = pl.pallas_call(
    _tc_pool_body,
    grid=(NJ,),
    in_specs=[pl.BlockSpec((B, RT, H), lambda j: (0, S_SC // RT + j, 0))],
    out_specs=pl.BlockSpec((B, 1, H), lambda j: (0, 0, 0)),
    out_shape=jax.ShapeDtypeStruct((B, 1, H), jnp.float32),
)


def _tail_body(ps_sc_ref, ps_tc_ref, w_ref, bias_ref, temp_ref, noise_ref,
               gum_ref, sel_ref, probs_ref):
    psum = ps_tc_ref[:, 0, :]
    for k in range(SSEG):
        psum = psum + ps_sc_ref[k]
    pooled = psum * (1.0 / S)
    w = w_ref[...]
    logits = lax.dot_general(pooled, w, (((1,), (0,)), ((), ())),
                             preferred_element_type=jnp.float32)
    logits = logits + bias_ref[...] + noise_ref[...]
    t = temp_ref[0, 0]
    x = logits / t
    x = x - jnp.max(x, axis=-1, keepdims=True)
    ex = jnp.exp(x)
    probs = ex / jnp.sum(ex, axis=-1, keepdims=True)
    probs_ref[...] = probs
    y = jnp.log(probs + 1e-20) + gum_ref[...]
    idx = lax.broadcasted_iota(jnp.int32, y.shape, 1)
    m1 = jnp.max(y, axis=-1, keepdims=True)
    i1 = jnp.min(jnp.where(y == m1, idx, E), axis=-1, keepdims=True)
    y2 = jnp.where(idx == i1, -jnp.inf, y)
    m2 = jnp.max(y2, axis=-1, keepdims=True)
    i2 = jnp.min(jnp.where(y2 == m2, idx, E), axis=-1, keepdims=True)
    sel_ref[...] = jnp.concatenate([i1, i2], axis=-1)


_tail = pl.pallas_call(
    _tail_body,
    out_shape=(jax.ShapeDtypeStruct((B, 2), jnp.int32),
               jax.ShapeDtypeStruct((B, E), jnp.float32)),
    in_specs=[
        pl.BlockSpec(memory_space=pltpu.VMEM),
        pl.BlockSpec(memory_space=pltpu.VMEM),
        pl.BlockSpec(memory_space=pltpu.VMEM),
        pl.BlockSpec(memory_space=pltpu.VMEM),
        pl.BlockSpec(memory_space=pltpu.SMEM),
        pl.BlockSpec(memory_space=pltpu.VMEM),
        pl.BlockSpec(memory_space=pltpu.VMEM),
    ],
)


def kernel(hidden_states, W, b, temperature, diversity_bonus):
    noise_key = jax.random.fold_in(jax.random.key(0), 1)
    noise = jax.random.normal(noise_key, (B, E), jnp.float32) * diversity_bonus
    g_key = jax.random.fold_in(jax.random.key(0), 2)
    gumbel = jax.random.gumbel(g_key, (B, E), jnp.float32)
    ps_sc = _pool(hidden_states)
    ps_tc = _tc_pool(hidden_states)
    sel, probs = _tail(ps_sc, ps_tc, W, b.reshape(1, E),
                       temperature.reshape(1, 1), noise, gumbel)
    return sel, probs
